# baseline (speedup 1.0000x reference)
import jax
import jax.numpy as jnp
from jax import lax
from jax.experimental import pallas as pl
from jax.experimental.pallas import tpu as pltpu
from jax.experimental.pallas import tpu_sc as plsc

ROWS = 256
PER_ROW = 3 * 224 * 224
LANE = 128
SUB = PER_ROW // LANE
CHUNKS = ROWS // 16

PIECES_PER_ROW = 7
PSUB = SUB // PIECES_PER_ROW
NPIECE = ROWS * PIECES_PER_ROW
NWORKER = 32
MAXP = NPIECE // NWORKER


def _gather16(vec, idx):
    return lax.gather(
        vec,
        idx[:, None],
        lax.GatherDimensionNumbers(
            offset_dims=(),
            collapsed_slice_dims=(0,),
            start_index_map=(0,),
        ),
        slice_sizes=(1,),
        mode=lax.GatherScatterMode.PROMISE_IN_BOUNDS,
    )


def _compact_body(y_hbm, perm_hbm, n_hbm, y_v, perm_v, n_v):
    cid = lax.axis_index("c")
    sid = lax.axis_index("s")

    @pl.when(jnp.logical_and(cid == 0, sid == 0))
    def _():
        pltpu.sync_copy(y_hbm, y_v)
        lane = lax.iota(jnp.int32, 16)
        last = jnp.full((16,), 15, jnp.int32)
        zero = jnp.zeros((16,), jnp.int32)
        one = jnp.full((16,), 1, jnp.int32)
        for i in range(CHUNKS):
            perm_v[pl.ds(i * 16, 16)] = zero
        base = zero
        for i in range(CHUNKS):
            yv = y_v[pl.ds(i * 16, 16)]
            m = yv == one
            mi = jnp.where(m, one, zero)
            c = plsc.cumsum(mi)
            pos = base + c - mi
            plsc.store_scatter(perm_v, [pos], lane + (i * 16), mask=m)
            base = base + _gather16(c, last)
        n_v[...] = base
        pltpu.sync_copy(perm_v, perm_hbm)
        pltpu.sync_copy(n_v, n_hbm)


def _reduce_body(r_hbm, x_hbm, perm_hbm, n_hbm, part_hbm,
                 perm_v, n_v, rbuf, xbuf, out_v, sems):
    cid = lax.axis_index("c")
    sid = lax.axis_index("s")
    w = sid * 2 + cid

    pltpu.sync_copy(perm_hbm, perm_v)
    pltpu.sync_copy(n_hbm, n_v)
    n = jnp.max(n_v[...])
    npieces = n * PIECES_PER_ROW
    m_w = lax.max(jnp.int32(0),
                  lax.min(jnp.int32(MAXP),
                          (npieces - w + NWORKER - 1) // NWORKER))

    def piece_of(t):
        j = w + t * NWORKER
        rowidx = lax.min(j // PIECES_PER_ROW, lax.max(n - 1, jnp.int32(0)))
        off = pl.multiple_of((rowidx // 16) * 16, 16)
        chunk = perm_v[pl.ds(off, 16)]
        lvec = jnp.full((16,), 1, jnp.int32) * lax.rem(rowidx, 16)
        row = jnp.max(_gather16(chunk, lvec))
        return row * PIECES_PER_ROW + lax.rem(j, PIECES_PER_ROW)

    def start(t):
        slot = lax.rem(t, 2)
        piece = piece_of(t)
        pltpu.make_async_copy(r_hbm.at[piece], rbuf.at[slot], sems.at[0, slot]).start()
        pltpu.make_async_copy(x_hbm.at[piece], xbuf.at[slot], sems.at[1, slot]).start()

    for s in range(2):
        @pl.when(s < m_w)
        def _():
            start(jnp.int32(s))

    zacc = jnp.zeros((16,), jnp.float32)

    def step(t, accs):
        slot = lax.rem(t, 2)
        piece = piece_of(t)
        pltpu.make_async_copy(r_hbm.at[piece], rbuf.at[slot], sems.at[0, slot]).wait()
        pltpu.make_async_copy(x_hbm.at[piece], xbuf.at[slot], sems.at[1, slot]).wait()

        def inner(qc, accs2):
            a = list(accs2)
            for s8 in range(8):
                q = qc * 8 + s8
                for u in range(LANE // 16):
                    rv = rbuf[slot, q, pl.ds(u * 16, 16)]
                    xv = xbuf[slot, q, pl.ds(u * 16, 16)]
                    a[u] = a[u] + lax.abs(rv - xv)
            return tuple(a)

        accs = lax.fori_loop(0, PSUB // 8, inner, accs)

        @pl.when(t + 2 < m_w)
        def _():
            start(t + 2)

        return accs

    accs = lax.fori_loop(0, m_w, step, (zacc,) * 8)
    acc = accs[0]
    for u in range(1, 8):
        acc = acc + accs[u]
    out_v[...] = acc
    pltpu.sync_copy(out_v, part_hbm.at[w])


_KERNEL_CACHE = {}


def _get_kernels():
    if "compact" not in _KERNEL_CACHE:
        mesh = plsc.VectorSubcoreMesh(core_axis_name="c", subcore_axis_name="s")
        params = pltpu.CompilerParams(needs_layout_passes=False)
        _KERNEL_CACHE["compact"] = pl.kernel(
            _compact_body,
            out_type=(
                jax.ShapeDtypeStruct((ROWS,), jnp.int32),
                jax.ShapeDtypeStruct((16,), jnp.int32),
            ),
            mesh=mesh,
            compiler_params=params,
            scratch_types=[
                pltpu.VMEM((ROWS,), jnp.int32),
                pltpu.VMEM((ROWS,), jnp.int32),
                pltpu.VMEM((16,), jnp.int32),
            ],
        )
        _KERNEL_CACHE["reduce"] = pl.kernel(
            _reduce_body,
            out_type=jax.ShapeDtypeStruct((NWORKER, 16), jnp.float32),
            mesh=mesh,
            compiler_params=params,
            scratch_types=[
                pltpu.VMEM((ROWS,), jnp.int32),
                pltpu.VMEM((16,), jnp.int32),
                pltpu.VMEM((2, PSUB, LANE), jnp.float32),
                pltpu.VMEM((2, PSUB, LANE), jnp.float32),
                pltpu.VMEM((16,), jnp.float32),
                pltpu.SemaphoreType.DMA((2, 2)),
            ],
        )
    return _KERNEL_CACHE["compact"], _KERNEL_CACHE["reduce"]


def kernel(recons, x, y):
    compact, reduce_k = _get_kernels()
    perm, nvec = compact(y)
    rp = recons.reshape(NPIECE, PSUB, LANE)
    xp = x.reshape(NPIECE, PSUB, LANE)
    part = reduce_k(rp, xp, perm, nvec)
    n = nvec[0]
    total = jnp.sum(part)
    denom = n.astype(jnp.float32) * jnp.float32(PER_ROW)
    return jnp.where(n > 0, total / denom, jnp.float32(0.0))


# speedup vs baseline: 2.1913x; 2.1913x over previous
import jax
import jax.numpy as jnp
from jax import lax
from jax.experimental import pallas as pl
from jax.experimental.pallas import tpu as pltpu
from jax.experimental.pallas import tpu_sc as plsc

ROWS = 256
PER_ROW = 3 * 224 * 224
LANE = 128
SUB = PER_ROW // LANE
CHUNKS = ROWS // 16

GRP = 8
NSLOT = 2


def _gather16(vec, idx):
    return lax.gather(
        vec,
        idx[:, None],
        lax.GatherDimensionNumbers(
            offset_dims=(),
            collapsed_slice_dims=(0,),
            start_index_map=(0,),
        ),
        slice_sizes=(1,),
        mode=lax.GatherScatterMode.PROMISE_IN_BOUNDS,
    )


def _compact_body(y_hbm, perm_hbm, n_hbm, y_v, perm_v, n_v):
    cid = lax.axis_index("c")
    sid = lax.axis_index("s")

    @pl.when(jnp.logical_and(cid == 0, sid == 0))
    def _():
        pltpu.sync_copy(y_hbm, y_v)
        lane = lax.iota(jnp.int32, 16)
        last = jnp.full((16,), 15, jnp.int32)
        zero = jnp.zeros((16,), jnp.int32)
        one = jnp.full((16,), 1, jnp.int32)
        for i in range(CHUNKS):
            perm_v[pl.ds(i * 16, 16)] = zero
        base = zero
        for i in range(CHUNKS):
            yv = y_v[pl.ds(i * 16, 16)]
            m = yv == one
            mi = jnp.where(m, one, zero)
            c = plsc.cumsum(mi)
            pos = base + c - mi
            plsc.store_scatter(perm_v, [pos], lane + (i * 16), mask=m)
            base = base + _gather16(c, last)
        n_v[...] = base
        pltpu.sync_copy(perm_v, perm_hbm)
        pltpu.sync_copy(n_v, n_hbm)


_COMPACT_CACHE = []


def _compact(y):
    if not _COMPACT_CACHE:
        _COMPACT_CACHE.append(
            pl.kernel(
                _compact_body,
                out_type=(
                    jax.ShapeDtypeStruct((ROWS,), jnp.int32),
                    jax.ShapeDtypeStruct((16,), jnp.int32),
                ),
                mesh=plsc.VectorSubcoreMesh(
                    core_axis_name="c", subcore_axis_name="s"
                ),
                compiler_params=pltpu.CompilerParams(needs_layout_passes=False),
                scratch_types=[
                    pltpu.VMEM((ROWS,), jnp.int32),
                    pltpu.VMEM((ROWS,), jnp.int32),
                    pltpu.VMEM((16,), jnp.int32),
                ],
            )
        )
    return _COMPACT_CACHE[0](y)


def _loss_body(perm_ref, n_ref, r_hbm, x_hbm, out_ref, rbuf, xbuf, acc, sems):
    n = n_ref[0]
    ngrp = (n + GRP - 1) // GRP

    def row_of(k):
        return perm_ref[lax.min(k, lax.max(n - 1, jnp.int32(0)))]

    def start(g):
        slot = lax.rem(g, NSLOT)
        for j in range(GRP):
            row = row_of(g * GRP + j)
            pltpu.make_async_copy(
                r_hbm.at[row], rbuf.at[slot, j], sems.at[0, slot, j]
            ).start()
            pltpu.make_async_copy(
                x_hbm.at[row], xbuf.at[slot, j], sems.at[1, slot, j]
            ).start()

    for s in range(NSLOT):
        @pl.when(s < ngrp)
        def _():
            start(jnp.int32(s))

    acc[...] = jnp.zeros((8, LANE), jnp.float32)

    def step(g, carry):
        slot = lax.rem(g, NSLOT)
        for j in range(GRP):
            row = row_of(g * GRP + j)
            pltpu.make_async_copy(
                r_hbm.at[row], rbuf.at[slot, j], sems.at[0, slot, j]
            ).wait()
            pltpu.make_async_copy(
                x_hbm.at[row], xbuf.at[slot, j], sems.at[1, slot, j]
            ).wait()
        part = jnp.zeros((8, LANE), jnp.float32)
        for j in range(GRP):
            w = (g * GRP + j < n).astype(jnp.float32)
            d = jnp.abs(rbuf[slot, j] - xbuf[slot, j])
            part = part + w * jnp.sum(d.reshape(SUB // 8, 8, LANE), axis=0)
        acc[...] += part

        @pl.when(g + NSLOT < ngrp)
        def _():
            start(g + NSLOT)

        return carry

    lax.fori_loop(0, ngrp, step, 0)
    total = jnp.sum(acc[...])
    denom = n.astype(jnp.float32) * jnp.float32(PER_ROW)
    out_ref[0, 0] = jnp.where(n > 0, total / denom, jnp.float32(0.0))


_loss = pl.pallas_call(
    _loss_body,
    grid_spec=pltpu.PrefetchScalarGridSpec(
        num_scalar_prefetch=2,
        grid=(1,),
        in_specs=[
            pl.BlockSpec(memory_space=pl.ANY),
            pl.BlockSpec(memory_space=pl.ANY),
        ],
        out_specs=pl.BlockSpec(memory_space=pltpu.SMEM),
        scratch_shapes=[
            pltpu.VMEM((NSLOT, GRP, SUB, LANE), jnp.float32),
            pltpu.VMEM((NSLOT, GRP, SUB, LANE), jnp.float32),
            pltpu.VMEM((8, LANE), jnp.float32),
            pltpu.SemaphoreType.DMA((2, NSLOT, GRP)),
        ],
    ),
    out_shape=jax.ShapeDtypeStruct((1, 1), jnp.float32),
)


def kernel(recons, x, y):
    perm, nvec = _compact(y)
    r3 = recons.reshape(ROWS, SUB, LANE)
    x3 = x.reshape(ROWS, SUB, LANE)
    out = _loss(perm, nvec, r3, x3)
    return out[0, 0]
